# speedup vs baseline: 1.2406x; 1.2406x over previous
import functools

import jax
import jax.numpy as jnp
from jax import lax
from jax.experimental import pallas as pl
from jax.experimental.pallas import tpu as pltpu
from jax.experimental.pallas import tpu_sc as plsc

MAX_LEN = 8192
EMBED = 1024
B_TOTAL = 4 * 4096

NC = 2
NS = 16
NW = NC * NS

B_PER_W = B_TOTAL // NW
CHUNK = 32
NCHUNK = B_PER_W // CHUNK


def _make_gather():
  mesh = plsc.VectorSubcoreMesh(core_axis_name="c", subcore_axis_name="s")

  @functools.partial(
      pl.kernel,
      mesh=mesh,
      out_type=jax.ShapeDtypeStruct((B_TOTAL, EMBED), jnp.float32),
      scratch_types=[
          pltpu.VMEM((B_PER_W,), jnp.int32),
          pltpu.VMEM((CHUNK, EMBED), jnp.float32),
          pltpu.SemaphoreType.DMA,
      ],
  )
  def gather_kernel(x_hbm, table_hbm, out_hbm, idx_v, rows_v, sem):
    wid = lax.axis_index("s") * NC + lax.axis_index("c")
    base = wid * B_PER_W
    pltpu.sync_copy(x_hbm.at[pl.ds(base, B_PER_W)], idx_v)
    for c in range(NCHUNK):
      pltpu.async_copy(
          table_hbm.at[idx_v.at[pl.ds(c * CHUNK, CHUNK)]], rows_v, sem
      ).wait()
      pltpu.sync_copy(rows_v, out_hbm.at[pl.ds(base + c * CHUNK, CHUNK)])

  return gather_kernel


_gather = _make_gather()


@jax.jit
def kernel(x, P):
  out = _gather(x.reshape(-1), P)
  return out.reshape(x.shape + (EMBED,))


# speedup vs baseline: 1.4516x; 1.1700x over previous
import functools

import jax
import jax.numpy as jnp
from jax import lax
from jax.experimental import pallas as pl
from jax.experimental.pallas import tpu as pltpu
from jax.experimental.pallas import tpu_sc as plsc

MAX_LEN = 8192
EMBED = 1024
B_TOTAL = 4 * 4096

NC = 2
NS = 16
NW = NC * NS

B_PER_W = B_TOTAL // NW
CHUNK = 32
NCHUNK = B_PER_W // CHUNK


def _make_gather():
  mesh = plsc.VectorSubcoreMesh(core_axis_name="c", subcore_axis_name="s")

  @functools.partial(
      pl.kernel,
      mesh=mesh,
      out_type=jax.ShapeDtypeStruct((B_TOTAL, EMBED), jnp.float32),
      scratch_types=[
          pltpu.VMEM((B_PER_W,), jnp.int32),
          pltpu.VMEM((CHUNK, EMBED), jnp.float32),
          pltpu.VMEM((CHUNK, EMBED), jnp.float32),
          pltpu.SemaphoreType.DMA,
          pltpu.SemaphoreType.DMA,
          pltpu.SemaphoreType.DMA,
          pltpu.SemaphoreType.DMA,
      ],
  )
  def gather_kernel(x_hbm, table_hbm, out_hbm, idx_v, rows0, rows1,
                    g0, g1, o0, o1):
    wid = lax.axis_index("s") * NC + lax.axis_index("c")
    base = wid * B_PER_W
    pltpu.sync_copy(x_hbm.at[pl.ds(base, B_PER_W)], idx_v)
    bufs = (rows0, rows1)
    gsems = (g0, g1)
    osems = (o0, o1)
    gathers = [None, None]
    writes = [None, None]
    gathers[0] = pltpu.async_copy(
        table_hbm.at[idx_v.at[pl.ds(0, CHUNK)]], bufs[0], gsems[0])
    for c in range(NCHUNK):
      cur = c % 2
      nxt = (c + 1) % 2
      if c + 1 < NCHUNK:
        if writes[nxt] is not None:
          writes[nxt].wait()
        gathers[nxt] = pltpu.async_copy(
            table_hbm.at[idx_v.at[pl.ds((c + 1) * CHUNK, CHUNK)]],
            bufs[nxt], gsems[nxt])
      gathers[cur].wait()
      writes[cur] = pltpu.async_copy(
          bufs[cur], out_hbm.at[pl.ds(base + c * CHUNK, CHUNK)], osems[cur])
    writes[0].wait()
    writes[1].wait()

  return gather_kernel


_gather = _make_gather()


@jax.jit
def kernel(x, P):
  out = _gather(x.reshape(-1), P)
  return out.reshape(x.shape + (EMBED,))


# speedup vs baseline: 1.4559x; 1.0030x over previous
import functools

import jax
import jax.numpy as jnp
from jax import lax
from jax.experimental import pallas as pl
from jax.experimental.pallas import tpu as pltpu
from jax.experimental.pallas import tpu_sc as plsc

MAX_LEN = 8192
EMBED = 1024
B_TOTAL = 4 * 4096

NC = 2
NS = 16
NW = NC * NS

B_PER_W = B_TOTAL // NW
CHUNK = 32
NCHUNK = B_PER_W // CHUNK
NBUF = 3


def _make_gather():
  mesh = plsc.VectorSubcoreMesh(core_axis_name="c", subcore_axis_name="s")

  @functools.partial(
      pl.kernel,
      mesh=mesh,
      out_type=jax.ShapeDtypeStruct((B_TOTAL, EMBED), jnp.float32),
      scratch_types=[
          pltpu.VMEM((B_PER_W,), jnp.int32),
      ]
      + [pltpu.VMEM((CHUNK, EMBED), jnp.float32)] * NBUF
      + [pltpu.SemaphoreType.DMA] * (2 * NBUF),
  )
  def gather_kernel(x_hbm, table_hbm, out_hbm, idx_v, *bufs_and_sems):
    bufs = bufs_and_sems[:NBUF]
    gsems = bufs_and_sems[NBUF:2 * NBUF]
    osems = bufs_and_sems[2 * NBUF:]
    wid = lax.axis_index("s") * NC + lax.axis_index("c")
    base = wid * B_PER_W
    pltpu.sync_copy(x_hbm.at[pl.ds(base, B_PER_W)], idx_v)

    def gather(c, b):
      return pltpu.async_copy(
          table_hbm.at[idx_v.at[pl.ds(c * CHUNK, CHUNK)]], bufs[b], gsems[b])

    def write(c, b):
      return pltpu.async_copy(
          bufs[b], out_hbm.at[pl.ds(base + c * CHUNK, CHUNK)], osems[b])

    gathers = [None] * NBUF
    writes = [None] * NBUF
    for c in range(min(NBUF, NCHUNK)):
      gathers[c] = gather(c, c)
    for c in range(NCHUNK):
      b = c % NBUF
      if c >= 1:
        p = c - 1 + NBUF
        if p < NCHUNK:
          bp = (c - 1) % NBUF
          writes[bp].wait()
          gathers[bp] = gather(p, bp)
      gathers[b].wait()
      writes[b] = write(c, b)
    for c in range(max(0, NCHUNK - NBUF), NCHUNK):
      writes[c % NBUF].wait()

  return gather_kernel


_gather = _make_gather()


@jax.jit
def kernel(x, P):
  out = _gather(x.reshape(-1), P)
  return out.reshape(x.shape + (EMBED,))
